# baseline (speedup 1.0000x reference)
import functools

import numpy as np
import jax
import jax.numpy as jnp
from jax import lax
from jax.experimental import pallas as pl
from jax.experimental.pallas import tpu as pltpu, tpu_sc as plsc

N_LEVELS = 16
F = 2
LOG2_HS = 19
BASE = 16
FINEST = 512
_b = np.exp((np.log(FINEST) - np.log(BASE)) / (N_LEVELS - 1))
RES = [int(BASE * _b ** i) for i in range(N_LEVELS)]
TSIZES = [min(2 ** LOG2_HS, r ** 3) for r in RES]
P2 = np.int32(np.uint32(2654435761).astype(np.int64) - (1 << 32))
P3 = np.int32(805459861)

N_POINTS = 1048576
NW = 32
PPW = N_POINTS // NW
C = 512
NG = C // 16
NF = N_LEVELS * F
NIDX = 16 * C
IDX_MINOR = 128
NDMA = NIDX // IDX_MINOR
NXD = 3 * C // IDX_MINOR
NOUT = NF * C
NOD = NOUT // IDX_MINOR
NCHUNK = PPW // C

_f32 = jnp.float32
_i32 = jnp.int32


def _mod_u32(c, m):
    if m & (m - 1) == 0:
        return jnp.bitwise_and(c, np.int32(m - 1))
    cf = c.astype(_f32)
    cf = jnp.where(c < 0, cf + np.float32(2.0 ** 32), cf)
    q = (cf * np.float32(1.0 / m)).astype(_i32)
    r = c - q * np.int32(m)
    r = jnp.where(r < 0, r + np.int32(m), r)
    r = jnp.where(r >= np.int32(m), r - np.int32(m), r)
    return r


def _axis_coords(v, res):
    p = v * np.float32(res)
    t = p.astype(_i32)
    frac = p - t.astype(_f32)
    i0 = jnp.where(t >= np.int32(res), t - np.int32(res), t)
    i1 = i0 + 1
    i1 = jnp.where(i1 == np.int32(res), 0, i1)
    return i0, i1, frac


def _sc_body(x_hbm, *rest):
    tbls = rest[:N_LEVELS]
    out_hbm = rest[N_LEVELS]
    (xs, ys, zs, wxa, wya, wza, wxb, wyb, wzb, xidx_v, idx_a, idx_b,
     rows_a, rows_b, out_v, oidx_v, sem, osem) = rest[N_LEVELS + 1:]
    wid = lax.axis_index("s") * 2 + lax.axis_index("c")
    iot = lax.iota(_i32, 16)

    idx_bufs = (idx_a, idx_b)
    row_bufs = (rows_a, rows_b)
    w_bufs = ((wxa, wya, wza), (wxb, wyb, wzb))

    def phase1(lvl, idx_v):
        res = RES[lvl]
        tsize = TSIZES[lvl]
        wx, wy, wz = w_bufs[lvl % 2]

        def p1_body(g, carry):
            s = g * 16
            ix0, ix1, fx = _axis_coords(xs[pl.ds(s, 16)], res)
            iy0, iy1, fy = _axis_coords(ys[pl.ds(s, 16)], res)
            iz0, iz1, fz = _axis_coords(zs[pl.ds(s, 16)], res)
            wx[pl.ds(s, 16)] = fx
            wy[pl.ds(s, 16)] = fy
            wz[pl.ds(s, 16)] = fz
            hy0 = iy0 * P2
            hy1 = iy1 * P2
            hz0 = iz0 * P3
            hz1 = iz1 * P3
            s00 = ix0 + hy0
            s01 = ix1 + hy0
            s10 = ix0 + hy1
            s11 = ix1 + hy1
            corners = (s00 + hz0, s00 + hz1, s10 + hz0, s10 + hz1,
                       s01 + hz0, s01 + hz1, s11 + hz0, s11 + hz1)
            grow = g // 8
            gcol = (g % 8) * 16
            for k in range(8):
                c = _mod_u32(corners[k], tsize)
                e0 = c + c
                idx_v[k * (2 * C // IDX_MINOR) + grow, pl.ds(gcol, 16)] = e0
                idx_v[k * (2 * C // IDX_MINOR) + (C // IDX_MINOR) + grow,
                      pl.ds(gcol, 16)] = e0 + 1
            return carry

        lax.fori_loop(0, NG, p1_body, 0)

    def fire(lvl, idx_v, rows_v):
        tbl = tbls[lvl]

        def f_body(j, carry):
            pltpu.make_async_copy(
                tbl.at[idx_v.at[j]],
                rows_v.at[pl.ds(j * IDX_MINOR, IDX_MINOR)],
                sem).start()
            return carry

        lax.fori_loop(0, NDMA, f_body, 0)

    def drain(lvl, idx_v, rows_v):
        tbl = tbls[lvl]

        def d_body(j, carry):
            pltpu.make_async_copy(
                tbl.at[idx_v.at[j]],
                rows_v.at[pl.ds(j * IDX_MINOR, IDX_MINOR)],
                sem).wait()
            return carry

        lax.fori_loop(0, NDMA, d_body, 0)

    def blend(lvl, rows_v):
        wx, wy, wz = w_bufs[lvl % 2]

        def p3_body(g, carry):
            s = g * 16
            fx = wx[pl.ds(s, 16)]
            fy = wy[pl.ds(s, 16)]
            fz = wz[pl.ds(s, 16)]
            gx = 1.0 - fx
            gy = 1.0 - fy
            gz = 1.0 - fz
            for j in range(F):
                f = [rows_v[pl.ds(k * (2 * C) + j * C + s, 16)]
                     for k in range(8)]
                a00 = f[0] * gx + f[4] * fx
                a01 = f[1] * gx + f[5] * fx
                a10 = f[2] * gx + f[6] * fx
                a11 = f[3] * gx + f[7] * fx
                b0 = a00 * gy + a10 * fy
                b1 = a01 * gy + a11 * fy
                out_v[pl.ds((2 * lvl + j) * C + s, 16)] = b0 * gz + b1 * fz
            return carry

        lax.fori_loop(0, NG, p3_body, 0)

    def chunk_body(t, carry):
        base = wid * PPW + t * C

        def xidx_body(g, carry):
            s = g * 16
            e = (base + s + iot) * 3
            grow = g // 8
            gcol = (g % 8) * 16
            for a in range(3):
                xidx_v[a * (C // IDX_MINOR) + grow, pl.ds(gcol, 16)] = e + a
            return carry

        lax.fori_loop(0, NG, xidx_body, 0)
        axes_v = (xs, ys, zs)
        for j in range(NXD):
            a, jj = j // (C // IDX_MINOR), j % (C // IDX_MINOR)
            pltpu.make_async_copy(
                x_hbm.at[xidx_v.at[j]],
                axes_v[a].at[pl.ds(jj * IDX_MINOR, IDX_MINOR)],
                sem).start()

        def oidx_body(r, carry):
            def og_body(g, carry2, r=r):
                s = g * 16
                tgt = (base + s + iot) * NF + r
                flat = r * C + s
                oidx_v[flat // IDX_MINOR, pl.ds(flat % IDX_MINOR, 16)] = tgt
                return carry2
            lax.fori_loop(0, NG, og_body, 0)
            return carry

        lax.fori_loop(0, NF, oidx_body, 0)

        for j in range(NXD):
            a, jj = j // (C // IDX_MINOR), j % (C // IDX_MINOR)
            pltpu.make_async_copy(
                x_hbm.at[xidx_v.at[j]],
                axes_v[a].at[pl.ds(jj * IDX_MINOR, IDX_MINOR)],
                sem).wait()

        phase1(0, idx_a)
        fire(0, idx_a, rows_a)
        for lvl in range(N_LEVELS):
            cur = idx_bufs[lvl % 2], row_bufs[lvl % 2]
            nxt = idx_bufs[(lvl + 1) % 2], row_bufs[(lvl + 1) % 2]
            if lvl + 1 < N_LEVELS:
                phase1(lvl + 1, nxt[0])
            drain(lvl, cur[0], cur[1])
            if lvl + 1 < N_LEVELS:
                fire(lvl + 1, nxt[0], nxt[1])
            blend(lvl, cur[1])

        def oscat(j, carry):
            pltpu.make_async_copy(
                out_v.at[pl.ds(j * IDX_MINOR, IDX_MINOR)],
                out_hbm.at[oidx_v.at[j]],
                osem).start()
            return carry

        lax.fori_loop(0, NOD, oscat, 0)

        def oscat_wait(j, carry):
            pltpu.make_async_copy(
                out_v.at[pl.ds(j * IDX_MINOR, IDX_MINOR)],
                out_hbm.at[oidx_v.at[j]],
                osem).wait()
            return carry

        lax.fori_loop(0, NOD, oscat_wait, 0)
        return carry

    lax.fori_loop(0, NCHUNK, chunk_body, 0)


@jax.jit
def _encode_sc(x_flat, *tbls):
    mesh = plsc.VectorSubcoreMesh(core_axis_name="c", subcore_axis_name="s",
                                  num_cores=2, num_subcores=16)
    f = pl.kernel(
        _sc_body,
        out_type=jax.ShapeDtypeStruct((N_POINTS * NF,), _f32),
        mesh=mesh,
        scratch_types=[
            pltpu.VMEM((C,), _f32),
            pltpu.VMEM((C,), _f32),
            pltpu.VMEM((C,), _f32),
            pltpu.VMEM((C,), _f32),
            pltpu.VMEM((C,), _f32),
            pltpu.VMEM((C,), _f32),
            pltpu.VMEM((C,), _f32),
            pltpu.VMEM((C,), _f32),
            pltpu.VMEM((C,), _f32),
            pltpu.VMEM((NXD, IDX_MINOR), _i32),
            pltpu.VMEM((NDMA, IDX_MINOR), _i32),
            pltpu.VMEM((NDMA, IDX_MINOR), _i32),
            pltpu.VMEM((NIDX,), _f32),
            pltpu.VMEM((NIDX,), _f32),
            pltpu.VMEM((NOUT,), _f32),
            pltpu.VMEM((NOD, IDX_MINOR), _i32),
            pltpu.SemaphoreType.DMA,
            pltpu.SemaphoreType.DMA,
        ],
    )
    return f(x_flat, *tbls).reshape(N_POINTS, NF)


def kernel(x, tables):
    return _encode_sc(x.reshape(-1), *(t.reshape(-1) for t in tables))
